# Initial kernel scaffold; baseline (speedup 1.0000x reference)
import functools

import jax
import jax.numpy as jnp
from jax import lax
from jax.experimental import pallas as pl
from jax.experimental.pallas import tpu as pltpu
from jax.experimental.pallas import tpu_sc as plsc

N = 10000
D = 128
E = 320000

NC = 2
NS = 16
NW = NC * NS

C = 128
CHUNKS = 79
E_PAD = NW * CHUNKS * C
DUMMY = N

NPAD = 10240
RPT = NPAD // NS
ROW_BLKS = RPT // C


def _sc_body(with_cnt, x_hbm, src_hbm, dst_hbm, za_hbm, zc_hbm, on_hbm,
             agg_out, cnt_out, sidx, didx, rows, onesb, zab, zcb,
             agg_sh, cnt_sh):
    c = lax.axis_index("c")
    s = lax.axis_index("s")
    wid = s * NC + c
    base = s * RPT

    pltpu.sync_copy(za_hbm, zab)
    pltpu.sync_copy(zab, agg_sh.at[pl.ds(base, RPT)])
    if with_cnt:
        pltpu.sync_copy(zc_hbm, zcb)
        pltpu.sync_copy(on_hbm, onesb)
        pltpu.sync_copy(zcb, cnt_sh.at[pl.ds(base, RPT)])
    plsc.subcore_barrier()

    def chunk(j, carry):
        pltpu.sync_copy(src_hbm.at[wid, j], sidx)
        pltpu.sync_copy(dst_hbm.at[wid, j], didx)
        pltpu.sync_copy(x_hbm.at[sidx], rows)
        pltpu.sync_copy(rows, agg_sh.at[didx], add=True)
        if with_cnt:
            pltpu.sync_copy(onesb, cnt_sh.at[didx], add=True)
        return carry

    lax.fori_loop(0, CHUNKS, chunk, 0)
    plsc.subcore_barrier()

    for k in range(ROW_BLKS):
        pltpu.sync_copy(agg_sh.at[pl.ds(base + k * C, C)], rows)
        pltpu.sync_copy(rows, agg_out.at[c, pl.ds(base + k * C, C)])
    if with_cnt:
        pltpu.sync_copy(cnt_sh.at[pl.ds(base, RPT)], zcb)
        pltpu.sync_copy(zcb, cnt_out.at[c, pl.ds(base, RPT)])


def _make_sc(with_cnt):
    mesh = plsc.VectorSubcoreMesh(core_axis_name="c", subcore_axis_name="s")
    out_type = [jax.ShapeDtypeStruct((NC, NPAD, D), jnp.float32)]
    if with_cnt:
        out_type.append(jax.ShapeDtypeStruct((NC, NPAD, 16), jnp.float32))
    scratch = [
        pltpu.VMEM((C,), jnp.int32),
        pltpu.VMEM((C,), jnp.int32),
        pltpu.VMEM((C, D), jnp.float32),
        pltpu.VMEM((C, 16), jnp.float32),
        pltpu.VMEM((RPT, D), jnp.float32),
        pltpu.VMEM((RPT, 16), jnp.float32),
        pltpu.VMEM_SHARED((NPAD, D), jnp.float32),
        pltpu.VMEM_SHARED((NPAD, 16), jnp.float32),
    ]
    if with_cnt:
        body = functools.partial(_sc_body, True)
    else:
        def body(x_hbm, src_hbm, dst_hbm, za_hbm, zc_hbm, on_hbm, agg_out,
                 *rest):
            return _sc_body(False, x_hbm, src_hbm, dst_hbm, za_hbm, zc_hbm,
                            on_hbm, agg_out, None, *rest)
    return pl.kernel(body, out_type=out_type, mesh=mesh,
                     scratch_types=scratch)


_sc_scatter_cnt = _make_sc(True)
_sc_scatter = _make_sc(False)


def _tc_body(x_ref, agg_ref, cnt_ref, wl_ref, wr_ref, b_ref, o_ref):
    a = agg_ref[0] + agg_ref[1]
    cn = cnt_ref[0, :, 0:1] + cnt_ref[1, :, 0:1]
    inv = 1.0 / jnp.maximum(cn, 1.0)
    a = a * inv
    acc = lax.dot_general(a, wl_ref[...], (((1,), (1,)), ((), ())),
                          preferred_element_type=jnp.float32)
    acc = acc + lax.dot_general(x_ref[...], wr_ref[...],
                                (((1,), (1,)), ((), ())),
                                preferred_element_type=jnp.float32)
    o_ref[...] = jnp.maximum(acc + b_ref[...], 0.0)


_TC_R = 1000


def _tc_layer(x, agg, cnt, Wl, Wr, b2d):
    grid = N // _TC_R
    return pl.pallas_call(
        _tc_body,
        grid=(grid,),
        in_specs=[
            pl.BlockSpec((_TC_R, D), lambda i: (i, 0)),
            pl.BlockSpec((NC, _TC_R, D), lambda i: (0, i, 0)),
            pl.BlockSpec((NC, _TC_R, 16), lambda i: (0, i, 0)),
            pl.BlockSpec((D, D), lambda i: (0, 0)),
            pl.BlockSpec((D, D), lambda i: (0, 0)),
            pl.BlockSpec((1, D), lambda i: (0, 0)),
        ],
        out_specs=pl.BlockSpec((_TC_R, D), lambda i: (i, 0)),
        out_shape=jax.ShapeDtypeStruct((N, D), jnp.float32),
    )(x, agg, cnt, Wl, Wr, b2d)


@jax.jit
def kernel(x, edge_index, W1_l, b1, W1_r, W2_l, b2, W2_r):
    src = edge_index[0].astype(jnp.int32)
    dst = edge_index[1].astype(jnp.int32)
    pad = E_PAD - E
    src = jnp.concatenate([src, jnp.zeros((pad,), jnp.int32)])
    dst = jnp.concatenate([dst, jnp.full((pad,), DUMMY, jnp.int32)])
    src = src.reshape(NW, CHUNKS, C)
    dst = dst.reshape(NW, CHUNKS, C)
    zeros_a = jnp.zeros((RPT, D), jnp.float32)
    zeros_c = jnp.zeros((RPT, 16), jnp.float32)
    ones_c = jnp.ones((C, 16), jnp.float32)

    agg1, cnt = _sc_scatter_cnt(x, src, dst, zeros_a, zeros_c, ones_c)
    h1 = _tc_layer(x, agg1, cnt, W1_l, W1_r, b1.reshape(1, D))
    (agg2,) = _sc_scatter(h1, src, dst, zeros_a, zeros_c, ones_c)
    h2 = _tc_layer(h1, agg2, cnt, W2_l, W2_r, b2.reshape(1, D))
    return h2


# speedup vs baseline: 4.1682x; 4.1682x over previous
import jax
import jax.numpy as jnp
from jax import lax
from jax.experimental import pallas as pl
from jax.experimental.pallas import tpu as pltpu
from jax.experimental.pallas import tpu_sc as plsc

N = 10000
D = 128
E = 320000

NC = 2
NS = 16
NW = NC * NS

C = 128
CHUNKS = 79
E_PAD = NW * CHUNKS * C
DUMMY = N

NPAD = 10240
RPT = NPAD // NS
ROW_BLKS = RPT // C

_MESH = plsc.VectorSubcoreMesh(core_axis_name="c", subcore_axis_name="s")


def _agg_body(x_hbm, src_hbm, dst_hbm, za_hbm, agg_out, sidx, didx, rows,
              agg_sh):
    c = lax.axis_index("c")
    s = lax.axis_index("s")
    wid = s * NC + c
    base = s * RPT

    pltpu.sync_copy(za_hbm, rows)
    for k in range(ROW_BLKS):
        pltpu.sync_copy(rows, agg_sh.at[pl.ds(base + k * C, C)])
    plsc.subcore_barrier()

    def chunk(j, carry):
        pltpu.sync_copy(src_hbm.at[wid, j], sidx)
        pltpu.sync_copy(dst_hbm.at[wid, j], didx)
        pltpu.sync_copy(x_hbm.at[sidx], rows)
        pltpu.sync_copy(rows, agg_sh.at[didx], add=True)
        return carry

    lax.fori_loop(0, CHUNKS, chunk, 0)
    plsc.subcore_barrier()

    for k in range(ROW_BLKS):
        pltpu.sync_copy(agg_sh.at[pl.ds(base + k * C, C)], rows)
        pltpu.sync_copy(rows, agg_out.at[c, pl.ds(base + k * C, C)])


def _cnt_body(dst_hbm, za_hbm, on_hbm, cnt_out, didx, rows, cnt_sh):
    c = lax.axis_index("c")
    s = lax.axis_index("s")
    wid = s * NC + c
    base = s * RPT

    pltpu.sync_copy(za_hbm, rows)
    for k in range(ROW_BLKS):
        pltpu.sync_copy(rows, cnt_sh.at[pl.ds(base + k * C, C)])
    pltpu.sync_copy(on_hbm, rows)
    plsc.subcore_barrier()

    def chunk(j, carry):
        pltpu.sync_copy(dst_hbm.at[wid, j], didx)
        pltpu.sync_copy(rows, cnt_sh.at[didx], add=True)
        return carry

    lax.fori_loop(0, CHUNKS, chunk, 0)
    plsc.subcore_barrier()

    for k in range(ROW_BLKS):
        pltpu.sync_copy(cnt_sh.at[pl.ds(base + k * C, C)], rows)
        pltpu.sync_copy(rows, cnt_out.at[c, pl.ds(base + k * C, C)])


_sc_agg = pl.kernel(
    _agg_body,
    out_type=jax.ShapeDtypeStruct((NC, NPAD, D), jnp.float32),
    mesh=_MESH,
    scratch_types=[
        pltpu.VMEM((C,), jnp.int32),
        pltpu.VMEM((C,), jnp.int32),
        pltpu.VMEM((C, D), jnp.float32),
        pltpu.VMEM_SHARED((NPAD, D), jnp.float32),
    ],
)

_sc_cnt = pl.kernel(
    _cnt_body,
    out_type=jax.ShapeDtypeStruct((NC, NPAD, D), jnp.float32),
    mesh=_MESH,
    scratch_types=[
        pltpu.VMEM((C,), jnp.int32),
        pltpu.VMEM((C, D), jnp.float32),
        pltpu.VMEM_SHARED((NPAD, D), jnp.float32),
    ],
)


def _tc_body(x_ref, agg_ref, cnt_ref, wl_ref, wr_ref, b_ref, o_ref):
    a = agg_ref[0] + agg_ref[1]
    cn = cnt_ref[0, :, 0:1] + cnt_ref[1, :, 0:1]
    inv = 1.0 / jnp.maximum(cn, 1.0)
    am = a * inv
    acc = lax.dot_general(am, wl_ref[...], (((1,), (1,)), ((), ())),
                          preferred_element_type=jnp.float32)
    acc = acc + lax.dot_general(x_ref[...], wr_ref[...],
                                (((1,), (1,)), ((), ())),
                                preferred_element_type=jnp.float32)
    o_ref[...] = jnp.maximum(acc + b_ref[...], 0.0)


_TC_R = 1000


def _tc_layer(x, agg, cnt, Wl, Wr, b2d):
    grid = N // _TC_R
    return pl.pallas_call(
        _tc_body,
        grid=(grid,),
        in_specs=[
            pl.BlockSpec((_TC_R, D), lambda i: (i, 0)),
            pl.BlockSpec((NC, _TC_R, D), lambda i: (0, i, 0)),
            pl.BlockSpec((NC, _TC_R, D), lambda i: (0, i, 0)),
            pl.BlockSpec((D, D), lambda i: (0, 0)),
            pl.BlockSpec((D, D), lambda i: (0, 0)),
            pl.BlockSpec((1, D), lambda i: (0, 0)),
        ],
        out_specs=pl.BlockSpec((_TC_R, D), lambda i: (i, 0)),
        out_shape=jax.ShapeDtypeStruct((N, D), jnp.float32),
    )(x, agg, cnt, Wl, Wr, b2d)


@jax.jit
def kernel(x, edge_index, W1_l, b1, W1_r, W2_l, b2, W2_r):
    src = edge_index[0].astype(jnp.int32)
    dst = edge_index[1].astype(jnp.int32)
    pad = E_PAD - E
    src = jnp.concatenate([src, jnp.zeros((pad,), jnp.int32)])
    dst = jnp.concatenate([dst, jnp.full((pad,), DUMMY, jnp.int32)])
    src = src.reshape(NW, CHUNKS, C)
    dst = dst.reshape(NW, CHUNKS, C)
    zeros_a = jnp.zeros((C, D), jnp.float32)
    ones_a = jnp.ones((C, D), jnp.float32)

    cnt = _sc_cnt(dst, zeros_a, ones_a)
    agg1 = _sc_agg(x, src, dst, zeros_a)
    h1 = _tc_layer(x, agg1, cnt, W1_l, W1_r, b1.reshape(1, D))
    agg2 = _sc_agg(h1, src, dst, zeros_a)
    h2 = _tc_layer(h1, agg2, cnt, W2_l, W2_r, b2.reshape(1, D))
    return h2
